# baseline (device time: 13969 ns/iter reference)
import functools

import jax
import jax.numpy as jnp
from jax import lax
from jax.experimental import pallas as pl
from jax.experimental.pallas import tpu as pltpu

N_DEV = 4
N_TOK = 256
D_IN = 128
D_OUT = 256
N_EXP = 8
EXP_PER = 2
ROWS_PER = N_TOK // N_DEV
N_HOP = N_DEV - 1


def kernel(x, router_W, route_idx, expert_W):
    def body(x_ref, rw_ref, idx_ref, ew_ref, out_ref,
             partial_ref, send_ref, recv_ref, send_sems, recv_sems):
        q = lax.axis_index("i")
        left = lax.rem(q + N_DEV - 1, N_DEV)
        right = lax.rem(q + 1, N_DEV)

        barrier_sem = pltpu.get_barrier_semaphore()
        for nbr in (left, right):
            pl.semaphore_signal(
                barrier_sem, inc=1,
                device_id=(nbr,), device_id_type=pl.DeviceIdType.MESH,
            )
        pl.semaphore_wait(barrier_sem, 2)

        xf = x_ref[:, :]
        scores = jnp.dot(xf, rw_ref[:, :], preferred_element_type=jnp.float32)
        s_max = jnp.max(scores, axis=-1, keepdims=True)
        e = jnp.exp(scores - s_max)
        probs = e / jnp.sum(e, axis=-1, keepdims=True)

        iota8 = lax.broadcasted_iota(jnp.int32, (N_TOK, N_EXP), 1)
        idx0 = idx_ref[:, 0:1]
        idx1 = idx_ref[:, 1:2]
        g0 = jnp.sum(jnp.where(iota8 == idx0, probs, 0.0), axis=-1,
                     keepdims=True)
        g1 = jnp.sum(jnp.where(iota8 == idx1, probs, 0.0), axis=-1,
                     keepdims=True)
        gs = g0 + g1

        e_base = q * EXP_PER
        acc = jnp.zeros((N_TOK, D_OUT), dtype=jnp.float32)
        for j in range(EXP_PER):
            eid = e_base + j
            gate = (jnp.where(idx0 == eid, g0, 0.0)
                    + jnp.where(idx1 == eid, g1, 0.0)) / gs
            xg = (xf * gate).astype(jnp.bfloat16)
            w = ew_ref[j, :, :].astype(jnp.bfloat16)
            acc = acc + jnp.dot(xg, w, preferred_element_type=jnp.float32)
        partial_ref[:, :] = acc

        def chunk(c):
            return partial_ref[pl.ds(c * ROWS_PER, ROWS_PER), :]

        send_ref[0, :, :] = chunk(lax.rem(q + N_DEV - 1, N_DEV))
        for h in range(N_HOP):
            rdma = pltpu.make_async_remote_copy(
                src_ref=send_ref.at[h],
                dst_ref=recv_ref.at[h],
                send_sem=send_sems.at[h],
                recv_sem=recv_sems.at[h],
                device_id=(right,),
                device_id_type=pl.DeviceIdType.MESH,
            )
            rdma.start()
            rdma.wait()
            c = lax.rem(q + 2 * N_DEV - 2 - h, N_DEV)
            val = recv_ref[h, :, :] + chunk(c)
            if h < N_HOP - 1:
                send_ref[h + 1, :, :] = val
            else:
                out_ref[:, :] = val

    return pl.pallas_call(
        body,
        out_shape=jax.ShapeDtypeStruct((ROWS_PER, D_OUT), jnp.float32),
        in_specs=[
            pl.BlockSpec(memory_space=pltpu.VMEM),
            pl.BlockSpec(memory_space=pltpu.VMEM),
            pl.BlockSpec(memory_space=pltpu.VMEM),
            pl.BlockSpec(memory_space=pltpu.VMEM),
        ],
        out_specs=pl.BlockSpec(memory_space=pltpu.VMEM),
        scratch_shapes=[
            pltpu.VMEM((N_TOK, D_OUT), jnp.float32),
            pltpu.VMEM((N_HOP, ROWS_PER, D_OUT), jnp.float32),
            pltpu.VMEM((N_HOP, ROWS_PER, D_OUT), jnp.float32),
            pltpu.SemaphoreType.DMA((N_HOP,)),
            pltpu.SemaphoreType.DMA((N_HOP,)),
        ],
        compiler_params=pltpu.CompilerParams(collective_id=0),
    )(x, router_W, route_idx, expert_W)


# device time: 9363 ns/iter; 1.4919x vs baseline; 1.4919x over previous
import jax
import jax.numpy as jnp
from jax import lax
from jax.experimental import pallas as pl
from jax.experimental.pallas import tpu as pltpu

N_DEV = 4
N_TOK = 256
D_IN = 128
D_OUT = 256
N_EXP = 8
EXP_PER = 2
ROWS_PER = N_TOK // N_DEV


def kernel(x, router_W, route_idx, expert_W):
    def body(x_ref, rw_ref, idx_ref, ew_ref, out_ref,
             pbf_ref, recv_ref, send_sems, recv_sems):
        q = lax.axis_index("i")

        barrier_sem = pltpu.get_barrier_semaphore()
        for k in range(1, N_DEV):
            pl.semaphore_signal(
                barrier_sem, inc=1,
                device_id=(lax.rem(q + k, N_DEV),),
                device_id_type=pl.DeviceIdType.MESH,
            )
        pl.semaphore_wait(barrier_sem, N_DEV - 1)

        xf = x_ref[:, :]
        scores = jnp.dot(xf, rw_ref[:, :], preferred_element_type=jnp.float32)
        s_max = jnp.max(scores, axis=-1, keepdims=True)
        e = jnp.exp(scores - s_max)
        probs = e / jnp.sum(e, axis=-1, keepdims=True)

        iota8 = lax.broadcasted_iota(jnp.int32, (N_TOK, N_EXP), 1)
        idx0 = idx_ref[:, 0:1]
        idx1 = idx_ref[:, 1:2]
        g0 = jnp.sum(jnp.where(iota8 == idx0, probs, 0.0), axis=-1,
                     keepdims=True)
        g1 = jnp.sum(jnp.where(iota8 == idx1, probs, 0.0), axis=-1,
                     keepdims=True)
        gs = g0 + g1

        e_base = q * EXP_PER
        acc = jnp.zeros((N_TOK, D_OUT), dtype=jnp.float32)
        for j in range(EXP_PER):
            eid = e_base + j
            gate = (jnp.where(idx0 == eid, g0, 0.0)
                    + jnp.where(idx1 == eid, g1, 0.0)) / gs
            xg = (xf * gate).astype(jnp.bfloat16)
            w = ew_ref[j, :, :].astype(jnp.bfloat16)
            acc = acc + jnp.dot(xg, w, preferred_element_type=jnp.float32)
        pbf_ref[:, :] = acc.astype(jnp.bfloat16)

        rdmas = []
        for k in range(1, N_DEV):
            d = lax.rem(q + k, N_DEV)
            s = N_DEV - 1 - k
            rdma = pltpu.make_async_remote_copy(
                src_ref=pbf_ref.at[pl.ds(d * ROWS_PER, ROWS_PER), :],
                dst_ref=recv_ref.at[s],
                send_sem=send_sems.at[s],
                recv_sem=recv_sems.at[s],
                device_id=(d,),
                device_id_type=pl.DeviceIdType.MESH,
            )
            rdma.start()
            rdmas.append(rdma)
        for rdma in rdmas:
            rdma.wait()

        own = pbf_ref[pl.ds(q * ROWS_PER, ROWS_PER), :].astype(jnp.float32)
        out_ref[:, :] = (own
                         + recv_ref[0, :, :].astype(jnp.float32)
                         + recv_ref[1, :, :].astype(jnp.float32)
                         + recv_ref[2, :, :].astype(jnp.float32))

    return pl.pallas_call(
        body,
        out_shape=jax.ShapeDtypeStruct((ROWS_PER, D_OUT), jnp.float32),
        in_specs=[
            pl.BlockSpec(memory_space=pltpu.VMEM),
            pl.BlockSpec(memory_space=pltpu.VMEM),
            pl.BlockSpec(memory_space=pltpu.VMEM),
            pl.BlockSpec(memory_space=pltpu.VMEM),
        ],
        out_specs=pl.BlockSpec(memory_space=pltpu.VMEM),
        scratch_shapes=[
            pltpu.VMEM((N_TOK, D_OUT), jnp.bfloat16),
            pltpu.VMEM((N_DEV - 1, ROWS_PER, D_OUT), jnp.bfloat16),
            pltpu.SemaphoreType.DMA((N_DEV - 1,)),
            pltpu.SemaphoreType.DMA((N_DEV - 1,)),
        ],
        compiler_params=pltpu.CompilerParams(collective_id=0),
    )(x, router_W, route_idx, expert_W)


# device time: 8558 ns/iter; 1.6323x vs baseline; 1.0941x over previous
import jax
import jax.numpy as jnp
from jax import lax
from jax.experimental import pallas as pl
from jax.experimental.pallas import tpu as pltpu

N_DEV = 4
N_TOK = 256
D_IN = 128
D_OUT = 256
N_EXP = 8
EXP_PER = 2
ROWS_PER = N_TOK // N_DEV


def kernel(x, router_W, route_idx, expert_W):
    def body(x_ref, rw_ref, idx_ref, ew_ref, out_ref,
             pbf_ref, recv_ref, send_sems, recv_sems):
        q = lax.axis_index("i")

        barrier_sem = pltpu.get_barrier_semaphore()
        for k in range(1, N_DEV):
            pl.semaphore_signal(
                barrier_sem, inc=1,
                device_id=(lax.rem(q + k, N_DEV),),
                device_id_type=pl.DeviceIdType.MESH,
            )

        xf = x_ref[:, :]
        scores = jnp.dot(xf, rw_ref[:, :], preferred_element_type=jnp.float32)
        s_max = jnp.max(scores, axis=-1, keepdims=True)
        e = jnp.exp(scores - s_max)
        probs = e / jnp.sum(e, axis=-1, keepdims=True)

        iota8 = lax.broadcasted_iota(jnp.int32, (N_TOK, N_EXP), 1)
        idx0 = idx_ref[:, 0:1]
        idx1 = idx_ref[:, 1:2]
        g0 = jnp.sum(jnp.where(iota8 == idx0, probs, 0.0), axis=-1,
                     keepdims=True)
        g1 = jnp.sum(jnp.where(iota8 == idx1, probs, 0.0), axis=-1,
                     keepdims=True)
        gs = g0 + g1

        e_base = q * EXP_PER
        acc = jnp.zeros((N_TOK, D_OUT), dtype=jnp.float32)
        for j in range(EXP_PER):
            eid = e_base + j
            gate = (jnp.where(idx0 == eid, g0, 0.0)
                    + jnp.where(idx1 == eid, g1, 0.0)) / gs
            xg = (xf * gate).astype(jnp.bfloat16)
            w = ew_ref[j, :, :].astype(jnp.bfloat16)
            acc = acc + jnp.dot(xg, w, preferred_element_type=jnp.float32)
        pbf_ref[:, :] = acc.astype(jnp.bfloat16)

        pl.semaphore_wait(barrier_sem, N_DEV - 1)

        rdmas = []
        for k in range(1, N_DEV):
            d = lax.rem(q + k, N_DEV)
            s = N_DEV - 1 - k
            rdma = pltpu.make_async_remote_copy(
                src_ref=pbf_ref.at[pl.ds(d * ROWS_PER, ROWS_PER), :],
                dst_ref=recv_ref.at[s],
                send_sem=send_sems.at[s],
                recv_sem=recv_sems.at[s],
                device_id=(d,),
                device_id_type=pl.DeviceIdType.MESH,
            )
            rdma.start()
            rdmas.append(rdma)

        own = pbf_ref[pl.ds(q * ROWS_PER, ROWS_PER), :].astype(jnp.float32)

        for rdma in rdmas:
            rdma.wait()

        out_ref[:, :] = (own
                         + recv_ref[0, :, :].astype(jnp.float32)
                         + recv_ref[1, :, :].astype(jnp.float32)
                         + recv_ref[2, :, :].astype(jnp.float32))

    return pl.pallas_call(
        body,
        out_shape=jax.ShapeDtypeStruct((ROWS_PER, D_OUT), jnp.float32),
        in_specs=[
            pl.BlockSpec(memory_space=pltpu.VMEM),
            pl.BlockSpec(memory_space=pltpu.VMEM),
            pl.BlockSpec(memory_space=pltpu.VMEM),
            pl.BlockSpec(memory_space=pltpu.VMEM),
        ],
        out_specs=pl.BlockSpec(memory_space=pltpu.VMEM),
        scratch_shapes=[
            pltpu.VMEM((N_TOK, D_OUT), jnp.bfloat16),
            pltpu.VMEM((N_DEV - 1, ROWS_PER, D_OUT), jnp.bfloat16),
            pltpu.SemaphoreType.DMA((N_DEV - 1,)),
            pltpu.SemaphoreType.DMA((N_DEV - 1,)),
        ],
        compiler_params=pltpu.CompilerParams(collective_id=0),
    )(x, router_W, route_idx, expert_W)
